# baseline (speedup 1.0000x reference)
import functools

import jax
import jax.numpy as jnp
from jax import lax
from jax.experimental import pallas as pl
from jax.experimental.pallas import tpu as pltpu
from jax.experimental.pallas import tpu_sc as plsc

T, D, E, HE, K = 2048, 2048, 8, 1024, 2
BLK = 128
NPAD = 5120
NB = NPAD // BLK
P2 = 2 * T


def _silu(x):
    return x * jax.nn.sigmoid(x)


def _route_body(x_ref, gw_ref, pos_ref, rowtok_ref, wvec_ref, blk_ref):
    x = x_ref[...]
    logits = jnp.dot(x, gw_ref[...], preferred_element_type=jnp.float32)
    m = jnp.max(logits, axis=1, keepdims=True)
    p = jnp.exp(logits - m)
    p = p / jnp.sum(p, axis=1, keepdims=True)
    ce = lax.broadcasted_iota(jnp.int32, (T, E), 1)
    v1 = jnp.max(p, axis=1, keepdims=True)
    i1 = jnp.min(jnp.where(p == v1, ce, E), axis=1, keepdims=True)
    p2 = jnp.where(ce == i1, -1.0, p)
    v2 = jnp.max(p2, axis=1, keepdims=True)
    i2 = jnp.min(jnp.where(p2 == v2, ce, E), axis=1, keepdims=True)

    C = ((ce == i1) | (ce == i2)).astype(jnp.float32)

    pres = []
    RB = 512
    for rb in range(T // RB):
        rowi = lax.broadcasted_iota(jnp.int32, (RB, T), 0) + RB * rb
        colj = lax.broadcasted_iota(jnp.int32, (RB, T), 1)
        tri = (colj < rowi).astype(jnp.float32)
        pres.append(jnp.dot(tri, C, preferred_element_type=jnp.float32))
    PRE = jnp.concatenate(pres, axis=0)

    cnt = jnp.sum(C, axis=0, keepdims=True)
    cntr = ((cnt.astype(jnp.int32) + (BLK - 1)) // BLK) * BLK
    eu = (lax.broadcasted_iota(jnp.int32, (E, E), 0)
          < lax.broadcasted_iota(jnp.int32, (E, E), 1)).astype(jnp.float32)
    po = jnp.dot(cntr.astype(jnp.float32), eu,
                 preferred_element_type=jnp.float32)

    base = po + PRE
    pos1 = jnp.sum(jnp.where(ce == i1, base, 0.0), axis=1, keepdims=True)
    pos2 = jnp.sum(jnp.where(ce == i2, base, 0.0), axis=1, keepdims=True)
    pos_ref[...] = jnp.concatenate([pos1, pos2], axis=1).astype(jnp.int32)

    p1i = pos1.astype(jnp.int32)
    p2i = pos2.astype(jnp.int32)
    tokcol = lax.broadcasted_iota(jnp.int32, (T, 1), 0).astype(jnp.float32)
    tdn = (((0,), (0,)), ((), ()))
    SC_ = 512
    for c in range(NPAD // SC_):
        srow = lax.broadcasted_iota(jnp.int32, (1, SC_), 1) + SC_ * c
        M1 = (p1i == srow).astype(jnp.float32)
        M2 = (p2i == srow).astype(jnp.float32)
        ones = jnp.ones((T, 1), jnp.float32)
        rt = (lax.dot_general(M1, tokcol, tdn, preferred_element_type=jnp.float32)
              + lax.dot_general(M2, tokcol, tdn, preferred_element_type=jnp.float32))
        wv = (lax.dot_general(M1, v1, tdn, preferred_element_type=jnp.float32)
              + lax.dot_general(M2, v2, tdn, preferred_element_type=jnp.float32))
        hit = (lax.dot_general(M1, ones, tdn, preferred_element_type=jnp.float32)
               + lax.dot_general(M2, ones, tdn, preferred_element_type=jnp.float32))
        scol = (lax.broadcasted_iota(jnp.int32, (SC_, 1), 0) + SC_ * c) % T
        rti = rt.astype(jnp.int32) + jnp.where(hit == 0.0, scol, 0)
        rowtok_ref[pl.ds(SC_ * c, SC_), :] = rti
        wvec_ref[pl.ds(SC_ * c, SC_), :] = wv

    ends = po + cntr.astype(jnp.float32)
    lane8 = lax.broadcasted_iota(jnp.int32, (1, E), 1)
    bvals = (BLK * lax.broadcasted_iota(jnp.int32, (1, 128), 1)).astype(jnp.float32)
    acc = jnp.zeros((1, 128), jnp.int32)
    for e in range(E):
        end_e = jnp.sum(jnp.where(lane8 == e, ends, 0.0))
        acc = acc + (bvals >= end_e).astype(jnp.int32)
    blkv = jnp.minimum(acc, E - 1)
    ptot = jnp.sum(jnp.where(lane8 == E - 1, ends, 0.0))
    nact = (ptot.astype(jnp.int32) + (BLK - 1)) // BLK
    lane128 = lax.broadcasted_iota(jnp.int32, (1, 128), 1)
    blk_ref[...] = jnp.where(lane128 == 64, nact, blkv)


def _route(x, gate_w):
    return pl.pallas_call(
        _route_body,
        out_shape=[
            jax.ShapeDtypeStruct((T, 2), jnp.int32),
            jax.ShapeDtypeStruct((NPAD, 1), jnp.int32),
            jax.ShapeDtypeStruct((NPAD, 1), jnp.float32),
            jax.ShapeDtypeStruct((1, 128), jnp.int32),
        ],
    )(x, gate_w)


def _sc_gather(table, idx, B):
    info = plsc.get_sparse_core_info()
    NW = info.num_cores * info.num_subcores
    b_per_w = B // NW
    CH = 16
    n_ch = b_per_w // CH
    mesh = plsc.VectorSubcoreMesh(core_axis_name="c", subcore_axis_name="s")

    @functools.partial(
        pl.kernel,
        mesh=mesh,
        out_type=jax.ShapeDtypeStruct((B, D), jnp.float32),
        scratch_types=[
            pltpu.VMEM((b_per_w,), jnp.int32),
            pltpu.VMEM((CH, D), jnp.float32),
            pltpu.VMEM((CH, D), jnp.float32),
            pltpu.SemaphoreType.DMA,
            pltpu.SemaphoreType.DMA,
            pltpu.SemaphoreType.DMA,
            pltpu.SemaphoreType.DMA,
        ],
    )
    def k(table_hbm, idx_hbm, out_hbm, idx_v, rows0, rows1, sg0, sg1, ss0, ss1):
        wid = lax.axis_index("s") * info.num_cores + lax.axis_index("c")
        base = wid * b_per_w
        pltpu.sync_copy(idx_hbm.at[pl.ds(base, b_per_w)], idx_v)
        bufs = (rows0, rows1)
        gsems = (sg0, sg1)
        ssems = (ss0, ss1)

        def gather(c):
            return pltpu.async_copy(
                table_hbm.at[idx_v.at[pl.ds(c * CH, CH)]], bufs[c % 2],
                gsems[c % 2])

        def store(c):
            return pltpu.make_async_copy(
                bufs[c % 2], out_hbm.at[pl.ds(base + c * CH, CH)],
                ssems[c % 2])

        stores = {}
        gathers = {0: gather(0)}
        for c in range(n_ch):
            if c + 1 < n_ch:
                if c - 1 >= 0:
                    stores[c - 1].wait()
                gathers[c + 1] = gather(c + 1)
            gathers[c].wait()
            stores[c] = store(c)
            stores[c].start()
        stores[n_ch - 1].wait()
        if n_ch >= 2:
            stores[n_ch - 2].wait()

    return k(table, idx)


def _gemm_body(be_ref, na_ref, xs_ref, wg_ref, wu_ref, wd_ref, bg_ref, bu_ref,
               bd_ref, wv_ref, o_ref):
    b = pl.program_id(0)

    @pl.when(b < na_ref[0])
    def _():
        xb = xs_ref[...].astype(jnp.bfloat16)
        g = jnp.dot(xb, wg_ref[0], preferred_element_type=jnp.float32) + bg_ref[0]
        u = jnp.dot(xb, wu_ref[0], preferred_element_type=jnp.float32) + bu_ref[0]
        hh = (_silu(g) * u).astype(jnp.bfloat16)
        y = jnp.dot(hh, wd_ref[0], preferred_element_type=jnp.float32) + bd_ref[0]
        o_ref[...] = y * wv_ref[...]


def _gemm(xs, Wg, Wu, Wd, bg3, bu3, bd3, wvec, blkexp, nactive):
    grid_spec = pltpu.PrefetchScalarGridSpec(
        num_scalar_prefetch=2,
        grid=(NB,),
        in_specs=[
            pl.BlockSpec((BLK, D), lambda b, be, na: (b, 0)),
            pl.BlockSpec((1, D, HE), lambda b, be, na: (be[b], 0, 0)),
            pl.BlockSpec((1, D, HE), lambda b, be, na: (be[b], 0, 0)),
            pl.BlockSpec((1, HE, D), lambda b, be, na: (be[b], 0, 0)),
            pl.BlockSpec((1, 1, HE), lambda b, be, na: (be[b], 0, 0)),
            pl.BlockSpec((1, 1, HE), lambda b, be, na: (be[b], 0, 0)),
            pl.BlockSpec((1, 1, D), lambda b, be, na: (be[b], 0, 0)),
            pl.BlockSpec((BLK, 1), lambda b, be, na: (b, 0)),
        ],
        out_specs=pl.BlockSpec((BLK, D), lambda b, be, na: (b, 0)),
    )
    return pl.pallas_call(
        _gemm_body,
        grid_spec=grid_spec,
        out_shape=jax.ShapeDtypeStruct((NPAD, D), jnp.float32),
    )(blkexp, nactive, xs, Wg, Wu, Wd, bg3, bu3, bd3, wvec)


TBC = 128


def _res_body(x_ref, wg_ref, wu_ref, wd_ref, bg_ref, bu_ref, bd_ref, o_ref):
    x = x_ref[...].astype(jnp.bfloat16)
    g = jnp.dot(x, wg_ref[...], preferred_element_type=jnp.float32) + bg_ref[...]
    u = jnp.dot(x, wu_ref[...], preferred_element_type=jnp.float32) + bu_ref[...]
    hh = (_silu(g) * u).astype(jnp.bfloat16)
    o_ref[...] = jnp.dot(hh, wd_ref[...], preferred_element_type=jnp.float32) + bd_ref[...]


def _res(x, rWg, rWu, rWd, rbg2, rbu2, rbd2):
    return pl.pallas_call(
        _res_body,
        grid=(T // TBC,),
        in_specs=[
            pl.BlockSpec((TBC, D), lambda t: (t, 0)),
            pl.BlockSpec((D, HE), lambda t: (0, 0)),
            pl.BlockSpec((D, HE), lambda t: (0, 0)),
            pl.BlockSpec((HE, D), lambda t: (0, 0)),
            pl.BlockSpec((1, HE), lambda t: (0, 0)),
            pl.BlockSpec((1, HE), lambda t: (0, 0)),
            pl.BlockSpec((1, D), lambda t: (0, 0)),
        ],
        out_specs=pl.BlockSpec((TBC, D), lambda t: (t, 0)),
        out_shape=jax.ShapeDtypeStruct((T, D), jnp.float32),
    )(x, rWg, rWu, rWd, rbg2, rbu2, rbd2)


def _add_body(res_ref, yp_ref, o_ref):
    o_ref[...] = res_ref[...] + yp_ref[:, 0, :] + yp_ref[:, 1, :]


def _add(res, ypair3):
    return pl.pallas_call(
        _add_body,
        grid=(T // 256,),
        in_specs=[
            pl.BlockSpec((256, D), lambda t: (t, 0)),
            pl.BlockSpec((256, 2, D), lambda t: (t, 0, 0)),
        ],
        out_specs=pl.BlockSpec((256, D), lambda t: (t, 0)),
        out_shape=jax.ShapeDtypeStruct((T, D), jnp.float32),
    )(res, ypair3)


def kernel(x, gate_w, Wg, Wu, Wd, bg, bu, bd, rgate_w, rWg, rWu, rWd, rbg, rbu, rbd):
    posP, rowtok, wvec, blk2d = _route(x, gate_w)
    blkexp = blk2d[0, :NB]
    nactive = blk2d[0, 64:65]
    Wgh, Wuh, Wdh = (w.astype(jnp.bfloat16) for w in (Wg, Wu, Wd))
    rWgh, rWuh, rWdh = (w.astype(jnp.bfloat16) for w in (rWg, rWu, rWd))
    res = _res(x, rWgh, rWuh, rWdh,
               rbg.reshape(1, HE), rbu.reshape(1, HE), rbd.reshape(1, D))
    xs = _sc_gather(x, rowtok.reshape(NPAD), NPAD)
    ys = _gemm(xs, Wgh, Wuh, Wdh,
               bg.reshape(E, 1, HE), bu.reshape(E, 1, HE), bd.reshape(E, 1, D),
               wvec, blkexp, nactive)
    ypair = _sc_gather(ys, posP.reshape(P2), P2)
    return _add(res, ypair.reshape(T, 2, D))


# speedup vs baseline: 1.2206x; 1.2206x over previous
import functools

import jax
import jax.numpy as jnp
from jax import lax
from jax.experimental import pallas as pl
from jax.experimental.pallas import tpu as pltpu
from jax.experimental.pallas import tpu_sc as plsc

T, D, E, HE, K = 2048, 2048, 8, 1024, 2
BLK = 128
NPAD = 5120
NB = NPAD // BLK
P2 = 2 * T


def _silu(x):
    return x * jax.nn.sigmoid(x)


def _route_body(x_ref, gw_ref, pos_ref, rowtok_ref, wvec_ref, blk_ref):
    x = x_ref[...]
    logits = jnp.dot(x, gw_ref[...], preferred_element_type=jnp.float32)
    m = jnp.max(logits, axis=1, keepdims=True)
    p = jnp.exp(logits - m)
    p = p / jnp.sum(p, axis=1, keepdims=True)
    ce = lax.broadcasted_iota(jnp.int32, (T, E), 1)
    v1 = jnp.max(p, axis=1, keepdims=True)
    i1 = jnp.min(jnp.where(p == v1, ce, E), axis=1, keepdims=True)
    p2 = jnp.where(ce == i1, -1.0, p)
    v2 = jnp.max(p2, axis=1, keepdims=True)
    i2 = jnp.min(jnp.where(p2 == v2, ce, E), axis=1, keepdims=True)

    C = ((ce == i1) | (ce == i2)).astype(jnp.float32)

    pres = []
    RB = 512
    for rb in range(T // RB):
        rowi = lax.broadcasted_iota(jnp.int32, (RB, T), 0) + RB * rb
        colj = lax.broadcasted_iota(jnp.int32, (RB, T), 1)
        tri = (colj < rowi).astype(jnp.float32)
        pres.append(jnp.dot(tri, C, preferred_element_type=jnp.float32))
    PRE = jnp.concatenate(pres, axis=0)

    cnt = jnp.sum(C, axis=0, keepdims=True)
    cntr = ((cnt.astype(jnp.int32) + (BLK - 1)) // BLK) * BLK
    eu = (lax.broadcasted_iota(jnp.int32, (E, E), 0)
          < lax.broadcasted_iota(jnp.int32, (E, E), 1)).astype(jnp.float32)
    po = jnp.dot(cntr.astype(jnp.float32), eu,
                 preferred_element_type=jnp.float32)

    base = po + PRE
    pos1 = jnp.sum(jnp.where(ce == i1, base, 0.0), axis=1, keepdims=True)
    pos2 = jnp.sum(jnp.where(ce == i2, base, 0.0), axis=1, keepdims=True)
    pos_ref[...] = jnp.concatenate([pos1, pos2], axis=1).astype(jnp.int32)

    p1i = pos1.astype(jnp.int32)
    p2i = pos2.astype(jnp.int32)
    tokcol = lax.broadcasted_iota(jnp.int32, (T, 1), 0).astype(jnp.float32)
    tdn = (((0,), (0,)), ((), ()))
    SC_ = 512
    for c in range(NPAD // SC_):
        srow = lax.broadcasted_iota(jnp.int32, (1, SC_), 1) + SC_ * c
        M1 = (p1i == srow).astype(jnp.float32)
        M2 = (p2i == srow).astype(jnp.float32)
        ones = jnp.ones((T, 1), jnp.float32)
        rt = (lax.dot_general(M1, tokcol, tdn, preferred_element_type=jnp.float32)
              + lax.dot_general(M2, tokcol, tdn, preferred_element_type=jnp.float32))
        wv = (lax.dot_general(M1, v1, tdn, preferred_element_type=jnp.float32)
              + lax.dot_general(M2, v2, tdn, preferred_element_type=jnp.float32))
        hit = (lax.dot_general(M1, ones, tdn, preferred_element_type=jnp.float32)
               + lax.dot_general(M2, ones, tdn, preferred_element_type=jnp.float32))
        scol = (lax.broadcasted_iota(jnp.int32, (SC_, 1), 0) + SC_ * c) % T
        rti = rt.astype(jnp.int32) + jnp.where(hit == 0.0, scol, 0)
        rowtok_ref[pl.ds(SC_ * c, SC_), :] = rti
        wvec_ref[pl.ds(SC_ * c, SC_), :] = wv

    ends = po + cntr.astype(jnp.float32)
    lane8 = lax.broadcasted_iota(jnp.int32, (1, E), 1)
    bvals = (BLK * lax.broadcasted_iota(jnp.int32, (1, 128), 1)).astype(jnp.float32)
    acc = jnp.zeros((1, 128), jnp.int32)
    for e in range(E):
        end_e = jnp.sum(jnp.where(lane8 == e, ends, 0.0))
        acc = acc + (bvals >= end_e).astype(jnp.int32)
    blkv = jnp.minimum(acc, E - 1)
    ptot = jnp.sum(jnp.where(lane8 == E - 1, ends, 0.0))
    nact = (ptot.astype(jnp.int32) + (BLK - 1)) // BLK
    lane128 = lax.broadcasted_iota(jnp.int32, (1, 128), 1)
    blk_ref[...] = jnp.where(lane128 == 64, nact, blkv)


def _route(x, gate_w):
    return pl.pallas_call(
        _route_body,
        out_shape=[
            jax.ShapeDtypeStruct((T, 2), jnp.int32),
            jax.ShapeDtypeStruct((NPAD, 1), jnp.int32),
            jax.ShapeDtypeStruct((NPAD, 1), jnp.float32),
            jax.ShapeDtypeStruct((1, 128), jnp.int32),
        ],
    )(x, gate_w)


def _sc_gather(table, idx, B):
    info = plsc.get_sparse_core_info()
    NW = info.num_cores * info.num_subcores
    b_per_w = B // NW
    CH = 16
    n_ch = b_per_w // CH
    mesh = plsc.VectorSubcoreMesh(core_axis_name="c", subcore_axis_name="s")

    @functools.partial(
        pl.kernel,
        mesh=mesh,
        out_type=jax.ShapeDtypeStruct((B, D), jnp.float32),
        scratch_types=[
            pltpu.VMEM((b_per_w,), jnp.int32),
            pltpu.VMEM((CH, D), jnp.float32),
            pltpu.VMEM((CH, D), jnp.float32),
            pltpu.SemaphoreType.DMA,
            pltpu.SemaphoreType.DMA,
            pltpu.SemaphoreType.DMA,
            pltpu.SemaphoreType.DMA,
        ],
    )
    def k(table_hbm, idx_hbm, out_hbm, idx_v, rows0, rows1, sg0, sg1, ss0, ss1):
        wid = lax.axis_index("s") * info.num_cores + lax.axis_index("c")
        base = wid * b_per_w
        pltpu.sync_copy(idx_hbm.at[pl.ds(base, b_per_w)], idx_v)
        bufs = (rows0, rows1)
        gsems = (sg0, sg1)
        ssems = (ss0, ss1)

        def gather(c):
            return pltpu.async_copy(
                table_hbm.at[idx_v.at[pl.ds(c * CH, CH)]], bufs[c % 2],
                gsems[c % 2])

        def store(c):
            return pltpu.make_async_copy(
                bufs[c % 2], out_hbm.at[pl.ds(base + c * CH, CH)],
                ssems[c % 2])

        stores = {}
        gathers = {0: gather(0)}
        for c in range(n_ch):
            if c + 1 < n_ch:
                if c - 1 >= 0:
                    stores[c - 1].wait()
                gathers[c + 1] = gather(c + 1)
            gathers[c].wait()
            stores[c] = store(c)
            stores[c].start()
        stores[n_ch - 1].wait()
        if n_ch >= 2:
            stores[n_ch - 2].wait()

    return k(table, idx)


def _gemm_body(be_ref, na_ref, xs_ref, wg_ref, wu_ref, wd_ref, bg_ref, bu_ref,
               bd_ref, wv_ref, o_ref):
    b = pl.program_id(0)

    @pl.when(b < na_ref[0])
    def _():
        xb = xs_ref[...]
        g = jnp.dot(xb, wg_ref[0], preferred_element_type=jnp.float32) + bg_ref[0]
        u = jnp.dot(xb, wu_ref[0], preferred_element_type=jnp.float32) + bu_ref[0]
        hh = _silu(g) * u
        y = jnp.dot(hh, wd_ref[0], preferred_element_type=jnp.float32) + bd_ref[0]
        o_ref[...] = y * wv_ref[...]


def _gemm(xs, Wg, Wu, Wd, bg3, bu3, bd3, wvec, blkexp, nactive):
    grid_spec = pltpu.PrefetchScalarGridSpec(
        num_scalar_prefetch=2,
        grid=(NB,),
        in_specs=[
            pl.BlockSpec((BLK, D), lambda b, be, na: (b, 0)),
            pl.BlockSpec((1, D, HE), lambda b, be, na: (be[b], 0, 0)),
            pl.BlockSpec((1, D, HE), lambda b, be, na: (be[b], 0, 0)),
            pl.BlockSpec((1, HE, D), lambda b, be, na: (be[b], 0, 0)),
            pl.BlockSpec((1, 1, HE), lambda b, be, na: (be[b], 0, 0)),
            pl.BlockSpec((1, 1, HE), lambda b, be, na: (be[b], 0, 0)),
            pl.BlockSpec((1, 1, D), lambda b, be, na: (be[b], 0, 0)),
            pl.BlockSpec((BLK, 1), lambda b, be, na: (b, 0)),
        ],
        out_specs=pl.BlockSpec((BLK, D), lambda b, be, na: (b, 0)),
    )
    return pl.pallas_call(
        _gemm_body,
        grid_spec=grid_spec,
        out_shape=jax.ShapeDtypeStruct((NPAD, D), jnp.float32),
    )(blkexp, nactive, xs, Wg, Wu, Wd, bg3, bu3, bd3, wvec)


TBC = 128


def _res_body(x_ref, wg_ref, wu_ref, wd_ref, bg_ref, bu_ref, bd_ref, o_ref):
    x = x_ref[...]
    g = jnp.dot(x, wg_ref[...], preferred_element_type=jnp.float32) + bg_ref[...]
    u = jnp.dot(x, wu_ref[...], preferred_element_type=jnp.float32) + bu_ref[...]
    hh = _silu(g) * u
    o_ref[...] = jnp.dot(hh, wd_ref[...], preferred_element_type=jnp.float32) + bd_ref[...]


def _res(x, rWg, rWu, rWd, rbg2, rbu2, rbd2):
    return pl.pallas_call(
        _res_body,
        grid=(T // TBC,),
        in_specs=[
            pl.BlockSpec((TBC, D), lambda t: (t, 0)),
            pl.BlockSpec((D, HE), lambda t: (0, 0)),
            pl.BlockSpec((D, HE), lambda t: (0, 0)),
            pl.BlockSpec((HE, D), lambda t: (0, 0)),
            pl.BlockSpec((1, HE), lambda t: (0, 0)),
            pl.BlockSpec((1, HE), lambda t: (0, 0)),
            pl.BlockSpec((1, D), lambda t: (0, 0)),
        ],
        out_specs=pl.BlockSpec((TBC, D), lambda t: (t, 0)),
        out_shape=jax.ShapeDtypeStruct((T, D), jnp.float32),
    )(x, rWg, rWu, rWd, rbg2, rbu2, rbd2)


def _add_body(res_ref, yp_ref, o_ref):
    o_ref[...] = res_ref[...] + yp_ref[:, 0, :] + yp_ref[:, 1, :]


def _add(res, ypair3):
    return pl.pallas_call(
        _add_body,
        grid=(T // 256,),
        in_specs=[
            pl.BlockSpec((256, D), lambda t: (t, 0)),
            pl.BlockSpec((256, 2, D), lambda t: (t, 0, 0)),
        ],
        out_specs=pl.BlockSpec((256, D), lambda t: (t, 0)),
        out_shape=jax.ShapeDtypeStruct((T, D), jnp.float32),
    )(res, ypair3)


def kernel(x, gate_w, Wg, Wu, Wd, bg, bu, bd, rgate_w, rWg, rWu, rWd, rbg, rbu, rbd):
    posP, rowtok, wvec, blk2d = _route(x, gate_w)
    blkexp = blk2d[0, :NB]
    nactive = blk2d[0, 64:65]
    res = _res(x, rWg, rWu, rWd,
               rbg.reshape(1, HE), rbu.reshape(1, HE), rbd.reshape(1, D))
    xs = _sc_gather(x, rowtok.reshape(NPAD), NPAD)
    ys = _gemm(xs, Wg, Wu, Wd,
               bg.reshape(E, 1, HE), bu.reshape(E, 1, HE), bd.reshape(E, 1, D),
               wvec, blkexp, nactive)
    ypair = _sc_gather(ys, posP.reshape(P2), P2)
    return _add(res, ypair.reshape(T, 2, D))


# speedup vs baseline: 1.3234x; 1.0842x over previous
import functools

import jax
import jax.numpy as jnp
from jax import lax
from jax.experimental import pallas as pl
from jax.experimental.pallas import tpu as pltpu
from jax.experimental.pallas import tpu_sc as plsc

T, D, E, HE, K = 2048, 2048, 8, 1024, 2
BLK = 128
NPAD = 5120
NB = NPAD // BLK
P2 = 2 * T


def _silu(x):
    return x * jax.nn.sigmoid(x)


def _route_body(x_ref, gw_ref, pos_ref, w_ref, blk_ref):
    x = x_ref[...]
    logits = jnp.dot(x, gw_ref[...], preferred_element_type=jnp.float32)
    m = jnp.max(logits, axis=1, keepdims=True)
    p = jnp.exp(logits - m)
    p = p / jnp.sum(p, axis=1, keepdims=True)
    ce = lax.broadcasted_iota(jnp.int32, (T, E), 1)
    v1 = jnp.max(p, axis=1, keepdims=True)
    i1 = jnp.min(jnp.where(p == v1, ce, E), axis=1, keepdims=True)
    p2 = jnp.where(ce == i1, -1.0, p)
    v2 = jnp.max(p2, axis=1, keepdims=True)
    i2 = jnp.min(jnp.where(p2 == v2, ce, E), axis=1, keepdims=True)

    C = ((ce == i1) | (ce == i2)).astype(jnp.float32)

    pres = []
    RB = 512
    for rb in range(T // RB):
        rowi = lax.broadcasted_iota(jnp.int32, (RB, T), 0) + RB * rb
        colj = lax.broadcasted_iota(jnp.int32, (RB, T), 1)
        tri = (colj < rowi).astype(jnp.float32)
        pres.append(jnp.dot(tri, C, preferred_element_type=jnp.float32))
    PRE = jnp.concatenate(pres, axis=0)

    cnt = jnp.sum(C, axis=0, keepdims=True)
    cntr = ((cnt.astype(jnp.int32) + (BLK - 1)) // BLK) * BLK
    eu = (lax.broadcasted_iota(jnp.int32, (E, E), 0)
          < lax.broadcasted_iota(jnp.int32, (E, E), 1)).astype(jnp.float32)
    po = jnp.dot(cntr.astype(jnp.float32), eu,
                 preferred_element_type=jnp.float32)

    base = po + PRE
    pos1 = jnp.sum(jnp.where(ce == i1, base, 0.0), axis=1, keepdims=True)
    pos2 = jnp.sum(jnp.where(ce == i2, base, 0.0), axis=1, keepdims=True)
    pos_ref[...] = jnp.concatenate([pos1, pos2], axis=1).astype(jnp.int32)

    w_ref[...] = jnp.concatenate([v1, v2], axis=1)

    ends = po + cntr.astype(jnp.float32)
    lane8 = lax.broadcasted_iota(jnp.int32, (1, E), 1)
    bvals = (BLK * lax.broadcasted_iota(jnp.int32, (1, 128), 1)).astype(jnp.float32)
    acc = jnp.zeros((1, 128), jnp.int32)
    for e in range(E):
        end_e = jnp.sum(jnp.where(lane8 == e, ends, 0.0))
        acc = acc + (bvals >= end_e).astype(jnp.int32)
    blkv = jnp.minimum(acc, E - 1)
    ptot = jnp.sum(jnp.where(lane8 == E - 1, ends, 0.0))
    nact = (ptot.astype(jnp.int32) + (BLK - 1)) // BLK
    lane128 = lax.broadcasted_iota(jnp.int32, (1, 128), 1)
    blk_ref[...] = jnp.where(lane128 == 64, nact, blkv)


def _route(x, gate_w):
    return pl.pallas_call(
        _route_body,
        out_shape=[
            jax.ShapeDtypeStruct((T, 2), jnp.int32),
            jax.ShapeDtypeStruct((T, 2), jnp.float32),
            jax.ShapeDtypeStruct((1, 128), jnp.int32),
        ],
    )(x, gate_w)


def _sc_scatter(posflat, wflat, initrt, zerosf):
    info = plsc.get_sparse_core_info()
    NS = info.num_subcores
    NW = info.num_cores * NS
    pairs_per_s = P2 // NS
    slots_per_s = NPAD // NS
    out_per_w = NPAD // NW
    mesh = plsc.VectorSubcoreMesh(core_axis_name="c", subcore_axis_name="s")

    @functools.partial(
        pl.kernel,
        mesh=mesh,
        out_type=[
            jax.ShapeDtypeStruct((NPAD,), jnp.int32),
            jax.ShapeDtypeStruct((NPAD,), jnp.float32),
        ],
        scratch_types=[
            pltpu.VMEM((pairs_per_s,), jnp.int32),
            pltpu.VMEM((pairs_per_s,), jnp.int32),
            pltpu.VMEM((pairs_per_s,), jnp.float32),
            pltpu.VMEM((slots_per_s,), jnp.int32),
            pltpu.VMEM((slots_per_s,), jnp.float32),
            pltpu.VMEM_SHARED((NPAD,), jnp.int32),
            pltpu.VMEM_SHARED((NPAD,), jnp.float32),
        ],
    )
    def k(pos_hbm, w_hbm, init_hbm, zc_hbm, rt_hbm, wv_hbm,
          pos_v, val_v, w_v, ibuf, fbuf, rt_sh, wv_sh):
        sid = lax.axis_index("s")
        cid = lax.axis_index("c")
        wid = sid * info.num_cores + cid
        sbase = sid * slots_per_s
        pltpu.sync_copy(init_hbm.at[pl.ds(sbase, slots_per_s)], ibuf)
        pltpu.sync_copy(ibuf, rt_sh.at[pl.ds(sbase, slots_per_s)])
        pltpu.sync_copy(zc_hbm.at[pl.ds(sbase, slots_per_s)], fbuf)
        pltpu.sync_copy(fbuf, wv_sh.at[pl.ds(sbase, slots_per_s)])
        pbase = sid * pairs_per_s
        pltpu.sync_copy(pos_hbm.at[pl.ds(pbase, pairs_per_s)], pos_v)
        pltpu.sync_copy(w_hbm.at[pl.ds(pbase, pairs_per_s)], w_v)
        for i in range(pairs_per_s // 16):
            pv = pos_v[pl.ds(i * 16, 16)]
            pidx = lax.broadcasted_iota(jnp.int32, (16,), 0) + (pbase + i * 16)
            tok = lax.shift_right_logical(pidx, 1)
            val_v[pl.ds(i * 16, 16)] = tok - (pv & (T - 1))
        plsc.subcore_barrier()
        pltpu.sync_copy(val_v, rt_sh.at[pos_v], add=True)
        pltpu.sync_copy(w_v, wv_sh.at[pos_v], add=True)
        plsc.subcore_barrier()
        obase = wid * out_per_w
        pltpu.sync_copy(rt_sh.at[pl.ds(obase, out_per_w)],
                        ibuf.at[pl.ds(0, out_per_w)])
        pltpu.sync_copy(ibuf.at[pl.ds(0, out_per_w)],
                        rt_hbm.at[pl.ds(obase, out_per_w)])
        pltpu.sync_copy(wv_sh.at[pl.ds(obase, out_per_w)],
                        fbuf.at[pl.ds(0, out_per_w)])
        pltpu.sync_copy(fbuf.at[pl.ds(0, out_per_w)],
                        wv_hbm.at[pl.ds(obase, out_per_w)])

    return k(posflat, wflat, initrt, zerosf)


def _sc_gather(table, idx, B):
    info = plsc.get_sparse_core_info()
    NW = info.num_cores * info.num_subcores
    b_per_w = B // NW
    CH = 16
    n_ch = b_per_w // CH
    mesh = plsc.VectorSubcoreMesh(core_axis_name="c", subcore_axis_name="s")

    @functools.partial(
        pl.kernel,
        mesh=mesh,
        out_type=jax.ShapeDtypeStruct((B, D), jnp.float32),
        scratch_types=[
            pltpu.VMEM((b_per_w,), jnp.int32),
            pltpu.VMEM((CH, D), jnp.float32),
            pltpu.VMEM((CH, D), jnp.float32),
            pltpu.SemaphoreType.DMA,
            pltpu.SemaphoreType.DMA,
            pltpu.SemaphoreType.DMA,
            pltpu.SemaphoreType.DMA,
        ],
    )
    def k(table_hbm, idx_hbm, out_hbm, idx_v, rows0, rows1, sg0, sg1, ss0, ss1):
        wid = lax.axis_index("s") * info.num_cores + lax.axis_index("c")
        base = wid * b_per_w
        pltpu.sync_copy(idx_hbm.at[pl.ds(base, b_per_w)], idx_v)
        bufs = (rows0, rows1)
        gsems = (sg0, sg1)
        ssems = (ss0, ss1)

        def gather(c):
            return pltpu.async_copy(
                table_hbm.at[idx_v.at[pl.ds(c * CH, CH)]], bufs[c % 2],
                gsems[c % 2])

        def store(c):
            return pltpu.make_async_copy(
                bufs[c % 2], out_hbm.at[pl.ds(base + c * CH, CH)],
                ssems[c % 2])

        stores = {}
        gathers = {0: gather(0)}
        for c in range(n_ch):
            if c + 1 < n_ch:
                if c - 1 >= 0:
                    stores[c - 1].wait()
                gathers[c + 1] = gather(c + 1)
            gathers[c].wait()
            stores[c] = store(c)
            stores[c].start()
        stores[n_ch - 1].wait()
        if n_ch >= 2:
            stores[n_ch - 2].wait()

    return k(table, idx)


def _gemm_body(be_ref, na_ref, xs_ref, wg_ref, wu_ref, wd_ref, bg_ref, bu_ref,
               bd_ref, wv_ref, o_ref):
    b = pl.program_id(0)

    @pl.when(b < na_ref[0])
    def _():
        xb = xs_ref[...]
        g = jnp.dot(xb, wg_ref[0], preferred_element_type=jnp.float32) + bg_ref[0]
        u = jnp.dot(xb, wu_ref[0], preferred_element_type=jnp.float32) + bu_ref[0]
        hh = _silu(g) * u
        y = jnp.dot(hh, wd_ref[0], preferred_element_type=jnp.float32) + bd_ref[0]
        o_ref[...] = y * wv_ref[...]


def _gemm(xs, Wg, Wu, Wd, bg3, bu3, bd3, wvec, blkexp, nactive):
    grid_spec = pltpu.PrefetchScalarGridSpec(
        num_scalar_prefetch=2,
        grid=(NB,),
        in_specs=[
            pl.BlockSpec((BLK, D), lambda b, be, na: (b, 0)),
            pl.BlockSpec((1, D, HE), lambda b, be, na: (be[b], 0, 0)),
            pl.BlockSpec((1, D, HE), lambda b, be, na: (be[b], 0, 0)),
            pl.BlockSpec((1, HE, D), lambda b, be, na: (be[b], 0, 0)),
            pl.BlockSpec((1, 1, HE), lambda b, be, na: (be[b], 0, 0)),
            pl.BlockSpec((1, 1, HE), lambda b, be, na: (be[b], 0, 0)),
            pl.BlockSpec((1, 1, D), lambda b, be, na: (be[b], 0, 0)),
            pl.BlockSpec((BLK, 1), lambda b, be, na: (b, 0)),
        ],
        out_specs=pl.BlockSpec((BLK, D), lambda b, be, na: (b, 0)),
    )
    return pl.pallas_call(
        _gemm_body,
        grid_spec=grid_spec,
        out_shape=jax.ShapeDtypeStruct((NPAD, D), jnp.float32),
    )(blkexp, nactive, xs, Wg, Wu, Wd, bg3, bu3, bd3, wvec)


TBC = 128


def _res_body(x_ref, wg_ref, wu_ref, wd_ref, bg_ref, bu_ref, bd_ref, o_ref):
    x = x_ref[...]
    g = jnp.dot(x, wg_ref[...], preferred_element_type=jnp.float32) + bg_ref[...]
    u = jnp.dot(x, wu_ref[...], preferred_element_type=jnp.float32) + bu_ref[...]
    hh = _silu(g) * u
    o_ref[...] = jnp.dot(hh, wd_ref[...], preferred_element_type=jnp.float32) + bd_ref[...]


def _res(x, rWg, rWu, rWd, rbg2, rbu2, rbd2):
    return pl.pallas_call(
        _res_body,
        grid=(T // TBC,),
        in_specs=[
            pl.BlockSpec((TBC, D), lambda t: (t, 0)),
            pl.BlockSpec((D, HE), lambda t: (0, 0)),
            pl.BlockSpec((D, HE), lambda t: (0, 0)),
            pl.BlockSpec((HE, D), lambda t: (0, 0)),
            pl.BlockSpec((1, HE), lambda t: (0, 0)),
            pl.BlockSpec((1, HE), lambda t: (0, 0)),
            pl.BlockSpec((1, D), lambda t: (0, 0)),
        ],
        out_specs=pl.BlockSpec((TBC, D), lambda t: (t, 0)),
        out_shape=jax.ShapeDtypeStruct((T, D), jnp.float32),
    )(x, rWg, rWu, rWd, rbg2, rbu2, rbd2)


def _add_body(res_ref, yp_ref, o_ref):
    o_ref[...] = res_ref[...] + yp_ref[:, 0, :] + yp_ref[:, 1, :]


def _add(res, ypair3):
    return pl.pallas_call(
        _add_body,
        grid=(T // 256,),
        in_specs=[
            pl.BlockSpec((256, D), lambda t: (t, 0)),
            pl.BlockSpec((256, 2, D), lambda t: (t, 0, 0)),
        ],
        out_specs=pl.BlockSpec((256, D), lambda t: (t, 0)),
        out_shape=jax.ShapeDtypeStruct((T, D), jnp.float32),
    )(res, ypair3)


def kernel(x, gate_w, Wg, Wu, Wd, bg, bu, bd, rgate_w, rWg, rWu, rWd, rbg, rbu, rbd):
    posP, wP, blk2d = _route(x, gate_w)
    blkexp = blk2d[0, :NB]
    nactive = blk2d[0, 64:65]
    initrt = jnp.arange(NPAD, dtype=jnp.int32) % T
    zerosf = jnp.zeros((NPAD,), jnp.float32)
    rowtok, wvec1 = _sc_scatter(posP.reshape(P2), wP.reshape(P2), initrt, zerosf)
    wvec = wvec1.reshape(NPAD, 1)
    res = _res(x, rWg, rWu, rWd,
               rbg.reshape(1, HE), rbu.reshape(1, HE), rbd.reshape(1, D))
    xs = _sc_gather(x, rowtok, NPAD)
    ys = _gemm(xs, Wg, Wu, Wd,
               bg.reshape(E, 1, HE), bu.reshape(E, 1, HE), bd.reshape(E, 1, D),
               wvec, blkexp, nactive)
    ypair = _sc_gather(ys, posP.reshape(P2), P2)
    return _add(res, ypair.reshape(T, 2, D))
